# baseline (speedup 1.0000x reference)
import jax
import jax.numpy as jnp
from jax.experimental import pallas as pl
from jax.experimental.pallas import tpu as pltpu


def _scale_kernel(x_ref, m_ref, o_ref):
    o_ref[...] = x_ref[...] * m_ref[...][None, :, None]


def kernel(x, mask):
    B, N, W = x.shape
    return pl.pallas_call(
        _scale_kernel,
        grid=(B,),
        in_specs=[
            pl.BlockSpec((1, N, W), lambda i: (i, 0, 0)),
            pl.BlockSpec((N,), lambda i: (0,)),
        ],
        out_specs=pl.BlockSpec((1, N, W), lambda i: (i, 0, 0)),
        out_shape=jax.ShapeDtypeStruct((B, N, W), x.dtype),
        compiler_params=pltpu.CompilerParams(
            dimension_semantics=("arbitrary",),
            vmem_limit_bytes=100 * 1024 * 1024,
        ),
    )(x, mask)
